# baseline (speedup 1.0000x reference)
import functools

import jax
import jax.numpy as jnp
from jax import lax
from jax.experimental import pallas as pl
from jax.experimental.pallas import tpu as pltpu
from jax.experimental.pallas import tpu_sc as plsc

L = 16
_B, _N, _S, _K = 4, 16384, 512, 16
_NCHUNK = _N // L
_MAX_IT = 64
_NW = 32
_SPW = (_B * _S) // _NW


def _vperm(v, idx):
    return v.at[idx].get(mode="promise_in_bounds")


def _vbcast(v, j):
    return _vperm(v, jnp.full((L,), j, dtype=jnp.int32))


def _as_scalar(v):
    return jnp.max(v) if getattr(v, "ndim", 0) else v


def _lane_sum_strided(v, iota):
    for h in (8, 4, 2, 1):
        v = v + _vperm(v, (iota + h) & (L - 1))
    return _vbcast(v, 0)


def _walk_body(pc_hbm, ctr_hbm, idx_hbm,
               pts_hbm, oidx_hbm, nbr_hbm, oit_hbm,
               pcx, pcy, pcz, ctx_, cty, ctz, idx_v,
               ptsx, ptsy, ptsz, oidx_v, nbr_v, oit_v, sem):
    cid = lax.axis_index("c")
    sid = lax.axis_index("s")
    wid = sid * 2 + cid
    b = wid // 8
    sb = (wid % 8) * _SPW

    pltpu.sync_copy(pc_hbm.at[pl.ds((b * 3 + 0) * _N, _N)], pcx)
    pltpu.sync_copy(pc_hbm.at[pl.ds((b * 3 + 1) * _N, _N)], pcy)
    pltpu.sync_copy(pc_hbm.at[pl.ds((b * 3 + 2) * _N, _N)], pcz)
    pltpu.sync_copy(ctr_hbm.at[pl.ds((b * 3 + 0) * _S, _S)], ctx_)
    pltpu.sync_copy(ctr_hbm.at[pl.ds((b * 3 + 1) * _S, _S)], cty)
    pltpu.sync_copy(ctr_hbm.at[pl.ds((b * 3 + 2) * _S, _S)], ctz)
    pltpu.sync_copy(idx_hbm.at[pl.ds(b * _S, _S)], idx_v)

    iota = jnp.arange(L, dtype=jnp.int32)
    lane0 = iota == 0

    def _sload(ref, i):
        return plsc.load_gather(ref, [jnp.full((L,), i, jnp.int32)])[0]

    def _sstore(ref, i, val):
        plsc.store_scatter(ref, [jnp.full((L,), i, jnp.int32)],
                           jnp.full((L,), val), mask=lane0)

    def dist_chunk(qxs, qys, qzs, base):
        px = pcx[pl.ds(base, L)]
        py = pcy[pl.ds(base, L)]
        pz = pcz[pl.ds(base, L)]
        dx = qxs - px
        dy = qys - py
        dz = qzs - pz
        return (dx * dx + dy * dy) + dz * dz

    def _fast_scalar(v):
        return v[0] if getattr(v, "ndim", 0) else v

    def _merge_chunk(d2c, base, carry):
        td, ti, tau = carry
        hit = _fast_scalar(plsc.all_reduce_population_count(d2c < tau)) > 0

        def merge(args):
            td, ti, _ = args
            kc, vc = plsc.sort_key_val(d2c, base + iota)
            rk = jnp.flip(kc)
            rv = jnp.flip(vc)
            m = td <= rk
            nd, ni = plsc.sort_key_val(jnp.where(m, td, rk),
                                       jnp.where(m, ti, rv))
            return nd, ni, _vbcast(nd, L - 1)

        return lax.cond(hit, merge, lambda a: a, (td, ti, tau))

    _GRP = 8

    _NS = 1

    def _load_slot(s):
        ss = jnp.minimum(s, _SPW - 1)
        return (_sload(ctx_, sb + ss), _sload(cty, sb + ss),
                _sload(ctz, sb + ss), _sload(idx_v, sb + ss))

    def w_cond(st):
        _, streams = st
        alive = streams[0][0]
        for k in range(1, _NS):
            alive = alive + streams[k][0]
        return alive > 0

    def w_body(st):
        nc, streams = st
        qs = [(jnp.full((L,), s[3]), jnp.full((L,), s[4]),
               jnp.full((L,), s[5])) for s in streams]

        tops = []
        for k in range(_NS):
            d0 = dist_chunk(*qs[k], 0)
            td, ti = plsc.sort_key_val(d0, iota)
            tk = (td, ti, _vbcast(td, L - 1))
            for c in range(1, _GRP):
                tk = _merge_chunk(dist_chunk(*qs[k], c * L), c * L, tk)
            tops.append(tk)
        tops = tuple(tops)

        def group(g, tops):
            base = g * (_GRP * L)
            d2s = [[dist_chunk(*qs[k], base + c * L) for c in range(_GRP)]
                   for k in range(_NS)]
            anyhit = None
            for k in range(_NS):
                mns = list(d2s[k])
                while len(mns) > 1:
                    mns = [jnp.minimum(mns[2 * i], mns[2 * i + 1])
                           for i in range(len(mns) // 2)]
                hm = mns[0] < tops[k][2]
                anyhit = hm if anyhit is None else (anyhit | hm)
            ghit = _fast_scalar(plsc.all_reduce_population_count(anyhit)) > 0

            def do_merges(tops):
                out = []
                for k in range(_NS):
                    tk = tops[k]
                    for c in range(_GRP):
                        tk = _merge_chunk(d2s[k][c], base + c * L, tk)
                    out.append(tk)
                return tuple(out)

            return lax.cond(ghit, do_merges, lambda a: a, tops)

        tops = plsc.parallel_loop(1, _NCHUNK // _GRP, unroll=4,
                                  carry=tops)(group)

        new_streams = []
        for k in range(_NS):
            alive, s_id, it, qx, qy, qz, si = streams[k]
            td, ti = tops[k][0], tops[k][1]

            nx = plsc.load_gather(pcx, [ti])
            ny = plsc.load_gather(pcy, [ti])
            nz = plsc.load_gather(pcz, [ti])
            cx = _lane_sum_strided(nx, iota) * jnp.float32(0.0625)
            cy = _lane_sum_strided(ny, iota) * jnp.float32(0.0625)
            cz = _lane_sum_strided(nz, iota) * jnp.float32(0.0625)
            ex = nx - cx
            ey = ny - cy
            ez = nz - cz
            cd2 = (ex * ex + ey * ey) + ez * ez

            c0 = _vbcast(cd2, 0)
            r0 = _fast_scalar(plsc.all_reduce_population_count(cd2 < c0))
            accept = jnp.logical_and(
                jnp.logical_or(r0 <= 4, it >= _MAX_IT), alive > 0)

            @pl.when(accept)
            def _(s_id=s_id, it=it, qx=qx, qy=qy, qz=qz, si=si, ti=ti):
                _sstore(ptsx, s_id, qx)
                _sstore(ptsy, s_id, qy)
                _sstore(ptsz, s_id, qz)
                _sstore(oidx_v, s_id, si)
                _sstore(oit_v, s_id, it)
                plsc.store_scatter(nbr_v, [s_id * _K + iota], ti)

            mn = jnp.min(cd2)
            jv = plsc.all_reduce_ffs(cd2 == jnp.full((L,), mn))
            jfull = jv if getattr(jv, "ndim", 0) == 1 else jnp.full((L,), jv)
            si_n = _vperm(ti, jfull)[0]
            qx_n = _sload(pcx, si_n)
            qy_n = _sload(pcy, si_n)
            qz_n = _sload(pcz, si_n)

            has = nc < _SPW
            qx_p, qy_p, qz_p, si_p = _load_slot(nc)
            pull = jnp.logical_and(accept, has)
            alive_n = jnp.where(alive > 0,
                                jnp.where(accept, pull.astype(jnp.int32),
                                          jnp.int32(1)),
                                jnp.int32(0))
            s_n = jnp.where(pull, nc, s_id)
            it_n = jnp.where(accept, jnp.int32(0), it + 1)
            new_streams.append((
                alive_n, s_n, it_n,
                jnp.where(accept, qx_p, qx_n),
                jnp.where(accept, qy_p, qy_n),
                jnp.where(accept, qz_p, qz_n),
                jnp.where(accept, si_p, si_n)))
            nc = nc + pull.astype(jnp.int32)

        return (nc, tuple(new_streams))

    init_streams = []
    for k in range(_NS):
        qx0, qy0, qz0, si0 = _load_slot(jnp.int32(k))
        init_streams.append((jnp.int32(1), jnp.int32(k), jnp.int32(0),
                             qx0, qy0, qz0, si0))
    lax.while_loop(w_cond, w_body, (jnp.int32(_NS), tuple(init_streams)))

    pltpu.sync_copy(ptsx, pts_hbm.at[pl.ds((b * 3 + 0) * _S + sb, _SPW)])
    pltpu.sync_copy(ptsy, pts_hbm.at[pl.ds((b * 3 + 1) * _S + sb, _SPW)])
    pltpu.sync_copy(ptsz, pts_hbm.at[pl.ds((b * 3 + 2) * _S + sb, _SPW)])
    pltpu.sync_copy(oidx_v, oidx_hbm.at[pl.ds(b * _S + sb, _SPW)])
    pltpu.sync_copy(nbr_v, nbr_hbm.at[pl.ds((b * _S + sb) * _K, _SPW * _K)])
    pltpu.sync_copy(oit_v, oit_hbm.at[pl.ds(b * _S + sb, _SPW)])


def kernel(xyz, center, idx):
    B, N, _ = xyz.shape
    S = center.shape[1]
    pc_t = jnp.transpose(xyz, (0, 2, 1)).reshape(-1)
    ctr_t = jnp.transpose(center, (0, 2, 1)).reshape(-1)
    idx_f = idx.reshape(-1)

    mesh = plsc.VectorSubcoreMesh(core_axis_name="c", subcore_axis_name="s")
    walk = functools.partial(
        pl.kernel,
        mesh=mesh,
        compiler_params=pltpu.CompilerParams(needs_layout_passes=False),
        out_type=(
            jax.ShapeDtypeStruct((B * 3 * S,), jnp.float32),
            jax.ShapeDtypeStruct((B * S,), jnp.int32),
            jax.ShapeDtypeStruct((B * S * _K,), jnp.int32),
            jax.ShapeDtypeStruct((B * S,), jnp.int32),
        ),
        scratch_types=[
            pltpu.VMEM((N,), jnp.float32),
            pltpu.VMEM((N,), jnp.float32),
            pltpu.VMEM((N,), jnp.float32),
            pltpu.VMEM((S,), jnp.float32),
            pltpu.VMEM((S,), jnp.float32),
            pltpu.VMEM((S,), jnp.float32),
            pltpu.VMEM((S,), jnp.int32),
            pltpu.VMEM((_SPW,), jnp.float32),
            pltpu.VMEM((_SPW,), jnp.float32),
            pltpu.VMEM((_SPW,), jnp.float32),
            pltpu.VMEM((_SPW,), jnp.int32),
            pltpu.VMEM((_SPW * _K,), jnp.int32),
            pltpu.VMEM((_SPW,), jnp.int32),
            pltpu.SemaphoreType.DMA,
        ],
    )(_walk_body)

    pts_f, oidx, nbr, oit = walk(pc_t, ctr_t, idx_f)

    pts = jnp.transpose(pts_f.reshape(B, 3, S), (0, 2, 1))
    oit = oit.reshape(B, S)
    order = jnp.argsort(oit, axis=1, stable=True)
    C = jnp.take_along_axis(pts, order[:, :, None], axis=1)
    I1 = jnp.take_along_axis(oidx.reshape(B, S), order, axis=1)
    I2 = jnp.take_along_axis(nbr.reshape(B, S, _K), order[:, :, None], axis=1)
    return (C, I1, I2)


# speedup vs baseline: 1.1024x; 1.1024x over previous
import functools

import jax
import jax.numpy as jnp
from jax import lax
from jax.experimental import pallas as pl
from jax.experimental.pallas import tpu as pltpu
from jax.experimental.pallas import tpu_sc as plsc

L = 16
_B, _N, _S, _K = 4, 16384, 512, 16
_NCHUNK = _N // L
_MAX_IT = 64
_NW = 32
_SPW = (_B * _S) // _NW


def _vperm(v, idx):
    return v.at[idx].get(mode="promise_in_bounds")


def _vbcast(v, j):
    return _vperm(v, jnp.full((L,), j, dtype=jnp.int32))


def _as_scalar(v):
    return jnp.max(v) if getattr(v, "ndim", 0) else v


def _lane_sum_strided(v, iota):
    for h in (8, 4, 2, 1):
        v = v + _vperm(v, (iota + h) & (L - 1))
    return _vbcast(v, 0)


def _walk_body(pc_hbm, ctr_hbm, idx_hbm,
               pts_hbm, oidx_hbm, nbr_hbm, oit_hbm,
               pcx, pcy, pcz, ctx_, cty, ctz, idx_v,
               ptsx, ptsy, ptsz, oidx_v, nbr_v, oit_v, sem):
    cid = lax.axis_index("c")
    sid = lax.axis_index("s")
    wid = sid * 2 + cid
    b = wid // 8
    sb = (wid % 8) * _SPW

    pltpu.sync_copy(pc_hbm.at[pl.ds((b * 3 + 0) * _N, _N)], pcx)
    pltpu.sync_copy(pc_hbm.at[pl.ds((b * 3 + 1) * _N, _N)], pcy)
    pltpu.sync_copy(pc_hbm.at[pl.ds((b * 3 + 2) * _N, _N)], pcz)
    pltpu.sync_copy(ctr_hbm.at[pl.ds((b * 3 + 0) * _S, _S)], ctx_)
    pltpu.sync_copy(ctr_hbm.at[pl.ds((b * 3 + 1) * _S, _S)], cty)
    pltpu.sync_copy(ctr_hbm.at[pl.ds((b * 3 + 2) * _S, _S)], ctz)
    pltpu.sync_copy(idx_hbm.at[pl.ds(b * _S, _S)], idx_v)

    iota = jnp.arange(L, dtype=jnp.int32)
    lane0 = iota == 0

    def _sload(ref, i):
        return plsc.load_gather(ref, [jnp.full((L,), i, jnp.int32)])[0]

    def _sstore(ref, i, val):
        plsc.store_scatter(ref, [jnp.full((L,), i, jnp.int32)],
                           jnp.full((L,), val), mask=lane0)

    def dist_chunk(qxs, qys, qzs, base):
        px = pcx[pl.ds(base, L)]
        py = pcy[pl.ds(base, L)]
        pz = pcz[pl.ds(base, L)]
        dx = qxs - px
        dy = qys - py
        dz = qzs - pz
        return (dx * dx + dy * dy) + dz * dz

    def _fast_scalar(v):
        return v[0] if getattr(v, "ndim", 0) else v

    def _merge_chunk(d2c, base, carry):
        td, ti, tau = carry
        hit = _fast_scalar(plsc.all_reduce_population_count(d2c < tau)) > 0

        def merge(args):
            td, ti, _ = args
            kc, vc = plsc.sort_key_val(d2c, base + iota)
            rk = jnp.flip(kc)
            rv = jnp.flip(vc)
            m = td <= rk
            nd, ni = plsc.sort_key_val(jnp.where(m, td, rk),
                                       jnp.where(m, ti, rv))
            return nd, ni, _vbcast(nd, L - 1)

        return lax.cond(hit, merge, lambda a: a, (td, ti, tau))

    _GRP = 4

    _NS = 1

    def _load_slot(s):
        ss = jnp.minimum(s, _SPW - 1)
        return (_sload(ctx_, sb + ss), _sload(cty, sb + ss),
                _sload(ctz, sb + ss), _sload(idx_v, sb + ss))

    def w_cond(st):
        _, streams = st
        alive = streams[0][0]
        for k in range(1, _NS):
            alive = alive + streams[k][0]
        return alive > 0

    def w_body(st):
        nc, streams = st
        qs = [(jnp.full((L,), s[3]), jnp.full((L,), s[4]),
               jnp.full((L,), s[5])) for s in streams]

        _SEED = 16
        tops = []
        for k in range(_NS):
            d0 = dist_chunk(*qs[k], 0)
            td, ti = plsc.sort_key_val(d0, iota)
            tk = (td, ti, _vbcast(td, L - 1))

            def seed(c, tk, k=k):
                return _merge_chunk(dist_chunk(*qs[k], c * L), c * L, tk)

            tk = lax.fori_loop(1, _SEED, seed, tk)
            tops.append(tk)
        tops = tuple(tops)

        def group(g, tops):
            base = g * (_GRP * L)
            d2s = [[dist_chunk(*qs[k], base + c * L) for c in range(_GRP)]
                   for k in range(_NS)]
            anyhit = None
            for k in range(_NS):
                mns = list(d2s[k])
                while len(mns) > 1:
                    mns = [jnp.minimum(mns[2 * i], mns[2 * i + 1])
                           for i in range(len(mns) // 2)]
                hm = mns[0] < tops[k][2]
                anyhit = hm if anyhit is None else (anyhit | hm)
            ghit = _fast_scalar(plsc.all_reduce_population_count(anyhit)) > 0

            def do_merges(tops):
                out = []
                for k in range(_NS):
                    tk = tops[k]
                    for c in range(_GRP):
                        tk = _merge_chunk(d2s[k][c], base + c * L, tk)
                    out.append(tk)
                return tuple(out)

            return lax.cond(ghit, do_merges, lambda a: a, tops)

        tops = plsc.parallel_loop(_SEED // _GRP, _NCHUNK // _GRP, unroll=8,
                                  carry=tops)(group)

        new_streams = []
        for k in range(_NS):
            alive, s_id, it, qx, qy, qz, si = streams[k]
            td, ti = tops[k][0], tops[k][1]

            nx = plsc.load_gather(pcx, [ti])
            ny = plsc.load_gather(pcy, [ti])
            nz = plsc.load_gather(pcz, [ti])
            cx = _lane_sum_strided(nx, iota) * jnp.float32(0.0625)
            cy = _lane_sum_strided(ny, iota) * jnp.float32(0.0625)
            cz = _lane_sum_strided(nz, iota) * jnp.float32(0.0625)
            ex = nx - cx
            ey = ny - cy
            ez = nz - cz
            cd2 = (ex * ex + ey * ey) + ez * ez

            c0 = _vbcast(cd2, 0)
            r0 = _fast_scalar(plsc.all_reduce_population_count(cd2 < c0))
            accept = jnp.logical_and(
                jnp.logical_or(r0 <= 4, it >= _MAX_IT), alive > 0)

            @pl.when(accept)
            def _(s_id=s_id, it=it, qx=qx, qy=qy, qz=qz, si=si, ti=ti):
                _sstore(ptsx, s_id, qx)
                _sstore(ptsy, s_id, qy)
                _sstore(ptsz, s_id, qz)
                _sstore(oidx_v, s_id, si)
                _sstore(oit_v, s_id, it)
                plsc.store_scatter(nbr_v, [s_id * _K + iota], ti)

            mn = jnp.min(cd2)
            jv = plsc.all_reduce_ffs(cd2 == jnp.full((L,), mn))
            jfull = jv if getattr(jv, "ndim", 0) == 1 else jnp.full((L,), jv)
            si_n = _vperm(ti, jfull)[0]
            qx_n = _sload(pcx, si_n)
            qy_n = _sload(pcy, si_n)
            qz_n = _sload(pcz, si_n)

            has = nc < _SPW
            qx_p, qy_p, qz_p, si_p = _load_slot(nc)
            pull = jnp.logical_and(accept, has)
            alive_n = jnp.where(alive > 0,
                                jnp.where(accept, pull.astype(jnp.int32),
                                          jnp.int32(1)),
                                jnp.int32(0))
            s_n = jnp.where(pull, nc, s_id)
            it_n = jnp.where(accept, jnp.int32(0), it + 1)
            new_streams.append((
                alive_n, s_n, it_n,
                jnp.where(accept, qx_p, qx_n),
                jnp.where(accept, qy_p, qy_n),
                jnp.where(accept, qz_p, qz_n),
                jnp.where(accept, si_p, si_n)))
            nc = nc + pull.astype(jnp.int32)

        return (nc, tuple(new_streams))

    init_streams = []
    for k in range(_NS):
        qx0, qy0, qz0, si0 = _load_slot(jnp.int32(k))
        init_streams.append((jnp.int32(1), jnp.int32(k), jnp.int32(0),
                             qx0, qy0, qz0, si0))
    lax.while_loop(w_cond, w_body, (jnp.int32(_NS), tuple(init_streams)))

    pltpu.sync_copy(ptsx, pts_hbm.at[pl.ds((b * 3 + 0) * _S + sb, _SPW)])
    pltpu.sync_copy(ptsy, pts_hbm.at[pl.ds((b * 3 + 1) * _S + sb, _SPW)])
    pltpu.sync_copy(ptsz, pts_hbm.at[pl.ds((b * 3 + 2) * _S + sb, _SPW)])
    pltpu.sync_copy(oidx_v, oidx_hbm.at[pl.ds(b * _S + sb, _SPW)])
    pltpu.sync_copy(nbr_v, nbr_hbm.at[pl.ds((b * _S + sb) * _K, _SPW * _K)])
    pltpu.sync_copy(oit_v, oit_hbm.at[pl.ds(b * _S + sb, _SPW)])


def kernel(xyz, center, idx):
    B, N, _ = xyz.shape
    S = center.shape[1]
    pc_t = jnp.transpose(xyz, (0, 2, 1)).reshape(-1)
    ctr_t = jnp.transpose(center, (0, 2, 1)).reshape(-1)
    idx_f = idx.reshape(-1)

    mesh = plsc.VectorSubcoreMesh(core_axis_name="c", subcore_axis_name="s")
    walk = functools.partial(
        pl.kernel,
        mesh=mesh,
        compiler_params=pltpu.CompilerParams(needs_layout_passes=False),
        out_type=(
            jax.ShapeDtypeStruct((B * 3 * S,), jnp.float32),
            jax.ShapeDtypeStruct((B * S,), jnp.int32),
            jax.ShapeDtypeStruct((B * S * _K,), jnp.int32),
            jax.ShapeDtypeStruct((B * S,), jnp.int32),
        ),
        scratch_types=[
            pltpu.VMEM((N,), jnp.float32),
            pltpu.VMEM((N,), jnp.float32),
            pltpu.VMEM((N,), jnp.float32),
            pltpu.VMEM((S,), jnp.float32),
            pltpu.VMEM((S,), jnp.float32),
            pltpu.VMEM((S,), jnp.float32),
            pltpu.VMEM((S,), jnp.int32),
            pltpu.VMEM((_SPW,), jnp.float32),
            pltpu.VMEM((_SPW,), jnp.float32),
            pltpu.VMEM((_SPW,), jnp.float32),
            pltpu.VMEM((_SPW,), jnp.int32),
            pltpu.VMEM((_SPW * _K,), jnp.int32),
            pltpu.VMEM((_SPW,), jnp.int32),
            pltpu.SemaphoreType.DMA,
        ],
    )(_walk_body)

    pts_f, oidx, nbr, oit = walk(pc_t, ctr_t, idx_f)

    pts = jnp.transpose(pts_f.reshape(B, 3, S), (0, 2, 1))
    oit = oit.reshape(B, S)
    order = jnp.argsort(oit, axis=1, stable=True)
    C = jnp.take_along_axis(pts, order[:, :, None], axis=1)
    I1 = jnp.take_along_axis(oidx.reshape(B, S), order, axis=1)
    I2 = jnp.take_along_axis(nbr.reshape(B, S, _K), order[:, :, None], axis=1)
    return (C, I1, I2)
